# baseline (speedup 1.0000x reference)
import jax
import jax.numpy as jnp
from jax import lax
from jax.experimental import pallas as pl
from jax.experimental.pallas import tpu as pltpu
from jax.experimental.pallas import tpu_sc as plsc

B, D, P, C = 8192, 1024, 1024, 16
NC0, NC1 = 8, 8
NCH = NC0 + NC1
BLK = 512

SC_CORES, SC_SUBCORES, SC_LANES = 2, 16, 16
NW = SC_CORES * SC_SUBCORES
B_PER_W = B // NW


def _tc_body(x_ref, wp_ref, bp_ref, ws_ref, bs_ref, wpc_ref, bpc_ref,
             wc0_ref, bc0_ref, wc1_ref, bc1_ref,
             pe_ref, pl_ref, ce_ref, d0_ref, d1_ref):
    dn = (((1,), (1,)), ((), ()))
    x = x_ref[...].astype(jnp.bfloat16)
    pe = lax.dot_general(x, wp_ref[...], dn,
                         preferred_element_type=jnp.float32) + bp_ref[...]
    pe_ref[...] = pe
    pl_ref[...] = lax.dot_general(pe.astype(jnp.bfloat16), wpc_ref[...], dn,
                                  preferred_element_type=jnp.float32) + bpc_ref[...]
    ce = lax.dot_general(x, ws_ref[...], dn,
                         preferred_element_type=jnp.float32) + bs_ref[...]
    ce_ref[...] = ce
    ce16 = ce.astype(jnp.bfloat16)
    d0_ref[...] = lax.dot_general(ce16, wc0_ref[...], dn,
                                  preferred_element_type=jnp.float32) + bc0_ref[...]
    d1_ref[...] = lax.dot_general(ce16, wc1_ref[...], dn,
                                  preferred_element_type=jnp.float32) + bc1_ref[...]


def _tc_call(x, Wp, bp, Ws, bs, Wpc, bpc, Wc0r, bc0r, Wc1r, bc1r):
    full = lambda shape: pl.BlockSpec(shape, lambda i: (0,) * len(shape))
    return pl.pallas_call(
        _tc_body,
        grid=(B // BLK,),
        in_specs=[
            pl.BlockSpec((BLK, D), lambda i: (i, 0)),
            full((P, D)), full((1, P)),
            full((P, D)), full((1, P)),
            full((C, P)), full((1, C)),
            full((C * NC0, P)), full((1, C * NC0)),
            full((C * NC1, P)), full((1, C * NC1)),
        ],
        out_specs=[
            pl.BlockSpec((BLK, P), lambda i: (i, 0)),
            pl.BlockSpec((BLK, C), lambda i: (i, 0)),
            pl.BlockSpec((BLK, P), lambda i: (i, 0)),
            pl.BlockSpec((BLK, C * NC0), lambda i: (i, 0)),
            pl.BlockSpec((BLK, C * NC1), lambda i: (i, 0)),
        ],
        out_shape=[
            jax.ShapeDtypeStruct((B, P), jnp.float32),
            jax.ShapeDtypeStruct((B, C), jnp.float32),
            jax.ShapeDtypeStruct((B, P), jnp.float32),
            jax.ShapeDtypeStruct((B, C * NC0), jnp.float32),
            jax.ShapeDtypeStruct((B, C * NC1), jnp.float32),
        ],
        compiler_params=pltpu.CompilerParams(
            dimension_semantics=("parallel",)),
    )(x, Wp, bp, Ws, bs, Wpc, bpc, Wc0r, bc0r, Wc1r, bc1r)


def _sc_body(d0_hbm, d1_hbm, y_hbm, out_hbm, y_v, chunk_v, out_v, sem):
    wid = lax.axis_index("s") * SC_CORES + lax.axis_index("c")
    base = wid * B_PER_W
    cp_y = pltpu.async_copy(y_hbm.at[pl.ds(base, B_PER_W)], y_v, sem)
    cp_0 = pltpu.async_copy(d0_hbm.at[pl.ds(base, B_PER_W)], chunk_v.at[0], sem)
    cp_1 = pltpu.async_copy(d1_hbm.at[pl.ds(base, B_PER_W)], chunk_v.at[1], sem)
    cp_y.wait()
    cp_0.wait()
    cp_1.wait()

    def group(g, carry):
        rows = g * SC_LANES + lax.iota(jnp.int32, SC_LANES)
        ycols = y_v[pl.ds(g * SC_LANES, SC_LANES)] * NC0
        for t in range(NCH):
            h, tt = divmod(t, NC0)
            val = plsc.load_gather(
                chunk_v, [jnp.full((SC_LANES,), h, jnp.int32), rows, ycols + tt])
            out_v[pl.ds(t * B_PER_W + g * SC_LANES, SC_LANES)] = val
        return carry

    lax.fori_loop(0, B_PER_W // SC_LANES, group, 0)
    pltpu.sync_copy(out_v, out_hbm.at[pl.ds(wid * NCH * B_PER_W, NCH * B_PER_W)])


def _sc_select(d0, d1, y):
    mesh = plsc.VectorSubcoreMesh(core_axis_name="c", subcore_axis_name="s",
                                  num_cores=SC_CORES, num_subcores=SC_SUBCORES)
    return pl.kernel(
        _sc_body,
        out_type=jax.ShapeDtypeStruct((NW * NCH * B_PER_W,), jnp.float32),
        mesh=mesh,
        scratch_types=[
            pltpu.VMEM((B_PER_W,), jnp.int32),
            pltpu.VMEM((2, B_PER_W, C * NC0), jnp.float32),
            pltpu.VMEM((NCH * B_PER_W,), jnp.float32),
            pltpu.SemaphoreType.DMA,
        ],
        compiler_params=pltpu.CompilerParams(use_tc_tiling_on_sc=False,
                                             needs_layout_passes=False),
    )(d0, d1, y)


def kernel(x, y, device, Wp, bp, Ws, bs, Wpc, bpc, Wc0, bc0, Wc1, bc1):
    bf = jnp.bfloat16
    pe, plog, ce, d0, d1 = _tc_call(
        x, Wp.astype(bf), bp.reshape(1, P), Ws.astype(bf), bs.reshape(1, P),
        Wpc.astype(bf), bpc.reshape(1, C),
        Wc0.reshape(C * NC0, P).astype(bf), bc0.reshape(1, C * NC0),
        Wc1.reshape(C * NC1, P).astype(bf), bc1.reshape(1, C * NC1))
    out = _sc_select(d0, d1, y.astype(jnp.int32))
    sel = out.reshape(NW, NCH, B_PER_W).transpose(0, 2, 1).reshape(B, NCH)
    child0 = sel[:, :NC0]
    child1 = sel[:, NC0:]
    return (plog, (child0, child1), pe, ce)


# speedup vs baseline: 1.0817x; 1.0817x over previous
import jax
import jax.numpy as jnp
from jax import lax
from jax.experimental import pallas as pl
from jax.experimental.pallas import tpu as pltpu
from jax.experimental.pallas import tpu_sc as plsc

B, D, P, C = 8192, 1024, 1024, 16
NC0, NC1 = 8, 8
NCH = NC0 + NC1
BLK = 512

SC_CORES, SC_SUBCORES, SC_LANES = 2, 16, 16
NW = SC_CORES * SC_SUBCORES
B_PER_W = B // NW


def _tc_body(x_ref, wp_ref, bp_ref, ws_ref, bs_ref, wpc_ref, bpc_ref,
             wc0_ref, bc0_ref, wc1_ref, bc1_ref,
             pe_ref, pl_ref, ce_ref, d0_ref, d1_ref):
    dn = (((1,), (1,)), ((), ()))
    x = x_ref[...]
    pe = lax.dot_general(x, wp_ref[...], dn,
                         preferred_element_type=jnp.float32) + bp_ref[...]
    pe_ref[...] = pe
    pl_ref[...] = lax.dot_general(pe, wpc_ref[...], dn,
                                  preferred_element_type=jnp.float32) + bpc_ref[...]
    ce = lax.dot_general(x, ws_ref[...], dn,
                         preferred_element_type=jnp.float32) + bs_ref[...]
    ce_ref[...] = ce
    d0_ref[...] = lax.dot_general(ce, wc0_ref[...], dn,
                                  preferred_element_type=jnp.float32) + bc0_ref[...]
    d1_ref[...] = lax.dot_general(ce, wc1_ref[...], dn,
                                  preferred_element_type=jnp.float32) + bc1_ref[...]


def _tc_call(x, Wp, bp, Ws, bs, Wpc, bpc, Wc0r, bc0r, Wc1r, bc1r):
    full = lambda shape: pl.BlockSpec(shape, lambda i: (0,) * len(shape))
    return pl.pallas_call(
        _tc_body,
        grid=(B // BLK,),
        in_specs=[
            pl.BlockSpec((BLK, D), lambda i: (i, 0)),
            full((P, D)), full((1, P)),
            full((P, D)), full((1, P)),
            full((C, P)), full((1, C)),
            full((C * NC0, P)), full((1, C * NC0)),
            full((C * NC1, P)), full((1, C * NC1)),
        ],
        out_specs=[
            pl.BlockSpec((BLK, P), lambda i: (i, 0)),
            pl.BlockSpec((BLK, C), lambda i: (i, 0)),
            pl.BlockSpec((BLK, P), lambda i: (i, 0)),
            pl.BlockSpec((BLK, C * NC0), lambda i: (i, 0)),
            pl.BlockSpec((BLK, C * NC1), lambda i: (i, 0)),
        ],
        out_shape=[
            jax.ShapeDtypeStruct((B, P), jnp.float32),
            jax.ShapeDtypeStruct((B, C), jnp.float32),
            jax.ShapeDtypeStruct((B, P), jnp.float32),
            jax.ShapeDtypeStruct((B, C * NC0), jnp.float32),
            jax.ShapeDtypeStruct((B, C * NC1), jnp.float32),
        ],
        compiler_params=pltpu.CompilerParams(
            dimension_semantics=("parallel",)),
    )(x, Wp, bp, Ws, bs, Wpc, bpc, Wc0r, bc0r, Wc1r, bc1r)


def _sc_body(d0_hbm, d1_hbm, y_hbm, out_hbm, y_v, chunk_v, out_v, sem):
    wid = lax.axis_index("s") * SC_CORES + lax.axis_index("c")
    base = wid * B_PER_W
    cp_y = pltpu.async_copy(y_hbm.at[pl.ds(base, B_PER_W)], y_v, sem)
    cp_0 = pltpu.async_copy(d0_hbm.at[pl.ds(base, B_PER_W)], chunk_v.at[0], sem)
    cp_1 = pltpu.async_copy(d1_hbm.at[pl.ds(base, B_PER_W)], chunk_v.at[1], sem)
    cp_y.wait()
    cp_0.wait()
    cp_1.wait()

    def group(g, carry):
        rows = g * SC_LANES + lax.iota(jnp.int32, SC_LANES)
        ycols = y_v[pl.ds(g * SC_LANES, SC_LANES)] * NC0
        for t in range(NCH):
            h, tt = divmod(t, NC0)
            val = plsc.load_gather(
                chunk_v, [jnp.full((SC_LANES,), h, jnp.int32), rows, ycols + tt])
            out_v[pl.ds(t * B_PER_W + g * SC_LANES, SC_LANES)] = val
        return carry

    lax.fori_loop(0, B_PER_W // SC_LANES, group, 0)
    pltpu.sync_copy(out_v, out_hbm.at[pl.ds(wid * NCH * B_PER_W, NCH * B_PER_W)])


def _sc_select(d0, d1, y):
    mesh = plsc.VectorSubcoreMesh(core_axis_name="c", subcore_axis_name="s",
                                  num_cores=SC_CORES, num_subcores=SC_SUBCORES)
    return pl.kernel(
        _sc_body,
        out_type=jax.ShapeDtypeStruct((NW * NCH * B_PER_W,), jnp.float32),
        mesh=mesh,
        scratch_types=[
            pltpu.VMEM((B_PER_W,), jnp.int32),
            pltpu.VMEM((2, B_PER_W, C * NC0), jnp.float32),
            pltpu.VMEM((NCH * B_PER_W,), jnp.float32),
            pltpu.SemaphoreType.DMA,
        ],
        compiler_params=pltpu.CompilerParams(use_tc_tiling_on_sc=False,
                                             needs_layout_passes=False),
    )(d0, d1, y)


def kernel(x, y, device, Wp, bp, Ws, bs, Wpc, bpc, Wc0, bc0, Wc1, bc1):
    pe, plog, ce, d0, d1 = _tc_call(
        x, Wp, bp.reshape(1, P), Ws, bs.reshape(1, P),
        Wpc, bpc.reshape(1, C),
        Wc0.reshape(C * NC0, P), bc0.reshape(1, C * NC0),
        Wc1.reshape(C * NC1, P), bc1.reshape(1, C * NC1))
    out = _sc_select(d0, d1, y.astype(jnp.int32))
    sel = out.reshape(NW, NCH, B_PER_W).transpose(0, 2, 1).reshape(B, NCH)
    child0 = sel[:, :NC0]
    child1 = sel[:, NC0:]
    return (plog, (child0, child1), pe, ce)


# speedup vs baseline: 1.1104x; 1.0265x over previous
import jax
import jax.numpy as jnp
from jax import lax
from jax.experimental import pallas as pl
from jax.experimental.pallas import tpu as pltpu
from jax.experimental.pallas import tpu_sc as plsc

B, D, P, C = 8192, 1024, 1024, 16
NC0, NC1 = 8, 8
NCH = NC0 + NC1
BLK = 1024

SC_CORES, SC_SUBCORES, SC_LANES = 2, 16, 16
NW = SC_CORES * SC_SUBCORES
B_PER_W = B // NW


def _tc_body(x_ref, wp_ref, bp_ref, ws_ref, bs_ref, wpc_ref, bpc_ref,
             wc0_ref, bc0_ref, wc1_ref, bc1_ref,
             pe_ref, pl_ref, ce_ref, d0_ref, d1_ref):
    dn = (((1,), (1,)), ((), ()))
    x = x_ref[...]
    pe = lax.dot_general(x, wp_ref[...], dn,
                         preferred_element_type=jnp.float32) + bp_ref[...]
    pe_ref[...] = pe
    pl_ref[...] = lax.dot_general(pe, wpc_ref[...], dn,
                                  preferred_element_type=jnp.float32) + bpc_ref[...]
    ce = lax.dot_general(x, ws_ref[...], dn,
                         preferred_element_type=jnp.float32) + bs_ref[...]
    ce_ref[...] = ce
    d0_ref[...] = lax.dot_general(ce, wc0_ref[...], dn,
                                  preferred_element_type=jnp.float32) + bc0_ref[...]
    d1_ref[...] = lax.dot_general(ce, wc1_ref[...], dn,
                                  preferred_element_type=jnp.float32) + bc1_ref[...]


def _tc_call(x, Wp, bp, Ws, bs, Wpc, bpc, Wc0r, bc0r, Wc1r, bc1r):
    full = lambda shape: pl.BlockSpec(shape, lambda i: (0,) * len(shape))
    return pl.pallas_call(
        _tc_body,
        grid=(B // BLK,),
        in_specs=[
            pl.BlockSpec((BLK, D), lambda i: (i, 0)),
            full((P, D)), full((1, P)),
            full((P, D)), full((1, P)),
            full((C, P)), full((1, C)),
            full((C * NC0, P)), full((1, C * NC0)),
            full((C * NC1, P)), full((1, C * NC1)),
        ],
        out_specs=[
            pl.BlockSpec((BLK, P), lambda i: (i, 0)),
            pl.BlockSpec((BLK, C), lambda i: (i, 0)),
            pl.BlockSpec((BLK, P), lambda i: (i, 0)),
            pl.BlockSpec((BLK, C * NC0), lambda i: (i, 0)),
            pl.BlockSpec((BLK, C * NC1), lambda i: (i, 0)),
        ],
        out_shape=[
            jax.ShapeDtypeStruct((B, P), jnp.float32),
            jax.ShapeDtypeStruct((B, C), jnp.float32),
            jax.ShapeDtypeStruct((B, P), jnp.float32),
            jax.ShapeDtypeStruct((B, C * NC0), jnp.float32),
            jax.ShapeDtypeStruct((B, C * NC1), jnp.float32),
        ],
        compiler_params=pltpu.CompilerParams(
            dimension_semantics=("parallel",)),
    )(x, Wp, bp, Ws, bs, Wpc, bpc, Wc0r, bc0r, Wc1r, bc1r)


def _sc_body(d0_hbm, d1_hbm, y_hbm, out_hbm, y_v, chunk_v, out_v, sem):
    wid = lax.axis_index("s") * SC_CORES + lax.axis_index("c")
    base = wid * B_PER_W
    cp_y = pltpu.async_copy(y_hbm.at[pl.ds(base, B_PER_W)], y_v, sem)
    cp_0 = pltpu.async_copy(d0_hbm.at[pl.ds(base, B_PER_W)], chunk_v.at[0], sem)
    cp_1 = pltpu.async_copy(d1_hbm.at[pl.ds(base, B_PER_W)], chunk_v.at[1], sem)
    cp_y.wait()
    cp_0.wait()
    cp_1.wait()

    def group(g, carry):
        rows = g * SC_LANES + lax.iota(jnp.int32, SC_LANES)
        ycols = y_v[pl.ds(g * SC_LANES, SC_LANES)] * NC0
        for t in range(NCH):
            h, tt = divmod(t, NC0)
            val = plsc.load_gather(
                chunk_v, [jnp.full((SC_LANES,), h, jnp.int32), rows, ycols + tt])
            out_v[pl.ds(t * B_PER_W + g * SC_LANES, SC_LANES)] = val
        return carry

    lax.fori_loop(0, B_PER_W // SC_LANES, group, 0)
    pltpu.sync_copy(out_v, out_hbm.at[pl.ds(wid * NCH * B_PER_W, NCH * B_PER_W)])


def _sc_select(d0, d1, y):
    mesh = plsc.VectorSubcoreMesh(core_axis_name="c", subcore_axis_name="s",
                                  num_cores=SC_CORES, num_subcores=SC_SUBCORES)
    return pl.kernel(
        _sc_body,
        out_type=jax.ShapeDtypeStruct((NW * NCH * B_PER_W,), jnp.float32),
        mesh=mesh,
        scratch_types=[
            pltpu.VMEM((B_PER_W,), jnp.int32),
            pltpu.VMEM((2, B_PER_W, C * NC0), jnp.float32),
            pltpu.VMEM((NCH * B_PER_W,), jnp.float32),
            pltpu.SemaphoreType.DMA,
        ],
        compiler_params=pltpu.CompilerParams(use_tc_tiling_on_sc=False,
                                             needs_layout_passes=False),
    )(d0, d1, y)


def kernel(x, y, device, Wp, bp, Ws, bs, Wpc, bpc, Wc0, bc0, Wc1, bc1):
    pe, plog, ce, d0, d1 = _tc_call(
        x, Wp, bp.reshape(1, P), Ws, bs.reshape(1, P),
        Wpc, bpc.reshape(1, C),
        Wc0.reshape(C * NC0, P), bc0.reshape(1, C * NC0),
        Wc1.reshape(C * NC1, P), bc1.reshape(1, C * NC1))
    out = _sc_select(d0, d1, y.astype(jnp.int32))
    sel = out.reshape(NW, NCH, B_PER_W).transpose(0, 2, 1).reshape(B, NCH)
    child0 = sel[:, :NC0]
    child1 = sel[:, NC0:]
    return (plog, (child0, child1), pe, ce)
